# baseline (speedup 1.0000x reference)
import functools

import jax
import jax.numpy as jnp
from jax import lax
from jax.experimental import pallas as pl
from jax.experimental.pallas import tpu as pltpu
from jax.experimental.pallas import tpu_sc as plsc

BN = 512


def _argmin_body(x_ref, cb_ref, a2_ref, b2_ref, idx_ref):
    x = x_ref[...]
    x2 = x + x
    cb = cb_ref[...]
    s2 = lax.dot_general(
        x2, cb, (((1,), (1,)), ((), ())),
        preferred_element_type=jnp.float32)
    d2 = (a2_ref[...] + b2_ref[...]) - s2
    k = d2.shape[1]
    m = jnp.min(d2, axis=1, keepdims=True)
    mc = jnp.maximum(m, 0.0)
    sm = jnp.sqrt(mc)
    base_bits = lax.bitcast_convert_type(sm * sm, jnp.int32) - 2
    count = jnp.zeros_like(base_bits)
    for db in range(6):
        cand = lax.bitcast_convert_type(base_bits + db, jnp.float32)
        count = count + (jnp.sqrt(cand) <= sm).astype(jnp.int32)
    t_bits = base_bits + count - 1
    t = lax.bitcast_convert_type(t_bits, jnp.float32)
    t = jnp.where(mc > 0.0, jnp.maximum(t, mc), 0.0)
    iota = lax.broadcasted_iota(jnp.int32, (1, k), 1).astype(jnp.float32)
    idx = jnp.min(jnp.where(d2 <= t, iota, float(k)), axis=1)
    idx_ref[...] = idx.astype(jnp.int32)


def _tc_argmin(flat, codebook, a2, b2):
    n, d = flat.shape
    k = codebook.shape[0]
    grid = (n // BN,)
    return pl.pallas_call(
        _argmin_body,
        grid=grid,
        in_specs=[
            pl.BlockSpec((BN, d), lambda i: (i, 0)),
            pl.BlockSpec((k, d), lambda i: (0, 0)),
            pl.BlockSpec((BN, 1), lambda i: (i, 0)),
            pl.BlockSpec((1, k), lambda i: (0, 0)),
        ],
        out_specs=pl.BlockSpec((BN,), lambda i: (i,)),
        out_shape=jax.ShapeDtypeStruct((n,), jnp.int32),
    )(flat, codebook, a2, b2)


_NC, _NS = 2, 16
_NW = _NC * _NS
_CHUNK = 96


_NBUF = 3


def _make_sc_gather(n, d):
    per_w = n // _NW
    n_chunks = per_w // _CHUNK
    mesh = plsc.VectorSubcoreMesh(core_axis_name="c", subcore_axis_name="s")

    @functools.partial(
        pl.kernel, mesh=mesh,
        out_type=jax.ShapeDtypeStruct((n, d), jnp.float32),
        scratch_types=[
            pltpu.VMEM((per_w,), jnp.int32),
        ] + [pltpu.VMEM((_CHUNK, d), jnp.float32) for _ in range(_NBUF)]
          + [pltpu.SemaphoreType.DMA for _ in range(2 * _NBUF)],
    )
    def gather(table_hbm, idx_hbm, out_hbm, idx_v, *bufs_and_sems):
        bufs = bufs_and_sems[:_NBUF]
        g_sem = bufs_and_sems[_NBUF:2 * _NBUF]
        w_sem = bufs_and_sems[2 * _NBUF:]
        wid = lax.axis_index("s") * _NC + lax.axis_index("c")
        base = wid * per_w
        pltpu.sync_copy(idx_hbm.at[pl.ds(base, per_w)], idx_v)

        gcp = [None] * n_chunks
        wcp = [None] * n_chunks

        def start_gather(c):
            b = c % _NBUF
            gcp[c] = pltpu.async_copy(
                table_hbm.at[idx_v.at[pl.ds(c * _CHUNK, _CHUNK)]],
                bufs[b], g_sem[b])

        for c in range(min(_NBUF, n_chunks)):
            start_gather(c)
        waited = [False] * n_chunks
        for c in range(n_chunks):
            b = c % _NBUF
            gcp[c].wait()
            wcp[c] = pltpu.async_copy(
                bufs[b], out_hbm.at[pl.ds(base + c * _CHUNK, _CHUNK)],
                w_sem[b])
            if c + _NBUF < n_chunks:
                wcp[c].wait()
                waited[c] = True
                start_gather(c + _NBUF)
        for c in range(n_chunks):
            if not waited[c]:
                wcp[c].wait()

    return gather


def kernel(input, codebook):
    batch_shape = input.shape[:-1]
    d = input.shape[-1]
    flat = input.reshape(-1, d)
    n = flat.shape[0]
    a2 = jnp.sum(flat * flat, axis=-1, keepdims=True)
    b2 = jnp.sum(codebook * codebook, axis=-1)[None, :]
    h = n // 2
    gather = _make_sc_gather(h, d)
    idx1 = _tc_argmin(flat[:h], codebook, a2[:h], b2)
    emb1 = gather(codebook, idx1)
    idx2 = _tc_argmin(flat[h:], codebook, a2[h:], b2)
    emb2 = gather(codebook, idx2)
    idx_flat = jnp.concatenate([idx1, idx2])
    embed = jnp.concatenate([emb1, emb2], axis=0)
    return embed.reshape(*batch_shape, d), idx_flat.reshape(batch_shape)


# speedup vs baseline: 1.0992x; 1.0992x over previous
import functools

import jax
import jax.numpy as jnp
from jax import lax
from jax.experimental import pallas as pl
from jax.experimental.pallas import tpu as pltpu
from jax.experimental.pallas import tpu_sc as plsc

BN = 512


def _argmin_body(x_ref, cb_ref, a2_ref, b2_ref, idx_ref):
    x = x_ref[...]
    x2 = x + x
    cb = cb_ref[...]
    s2 = lax.dot_general(
        x2, cb, (((1,), (1,)), ((), ())),
        preferred_element_type=jnp.float32)
    d2 = (a2_ref[...] + b2_ref[...]) - s2
    k = d2.shape[1]
    m = jnp.min(d2, axis=1, keepdims=True)
    mc = jnp.maximum(m, 0.0)
    sm = jnp.sqrt(mc)
    base_bits = lax.bitcast_convert_type(sm * sm, jnp.int32) - 2
    count = jnp.zeros_like(base_bits)
    for db in range(6):
        cand = lax.bitcast_convert_type(base_bits + db, jnp.float32)
        count = count + (jnp.sqrt(cand) <= sm).astype(jnp.int32)
    t_bits = base_bits + count - 1
    t = lax.bitcast_convert_type(t_bits, jnp.float32)
    t = jnp.where(mc > 0.0, jnp.maximum(t, mc), 0.0)
    iota = lax.broadcasted_iota(jnp.int32, (1, k), 1).astype(jnp.float32)
    idx = jnp.min(jnp.where(d2 <= t, iota, float(k)), axis=1)
    idx_ref[...] = idx.astype(jnp.int32)


def _tc_argmin(flat, codebook, a2, b2):
    n, d = flat.shape
    k = codebook.shape[0]
    grid = (n // BN,)
    return pl.pallas_call(
        _argmin_body,
        grid=grid,
        in_specs=[
            pl.BlockSpec((BN, d), lambda i: (i, 0)),
            pl.BlockSpec((k, d), lambda i: (0, 0)),
            pl.BlockSpec((BN, 1), lambda i: (i, 0)),
            pl.BlockSpec((1, k), lambda i: (0, 0)),
        ],
        out_specs=pl.BlockSpec((BN,), lambda i: (i,)),
        out_shape=jax.ShapeDtypeStruct((n,), jnp.int32),
    )(flat, codebook, a2, b2)


_NC, _NS = 2, 16
_NW = _NC * _NS
_CHUNK = 96


_NBUF = 3


def _make_sc_gather(n, d):
    per_w = n // _NW
    n_chunks = per_w // _CHUNK
    mesh = plsc.VectorSubcoreMesh(core_axis_name="c", subcore_axis_name="s")

    @functools.partial(
        pl.kernel, mesh=mesh,
        out_type=jax.ShapeDtypeStruct((n, d), jnp.float32),
        scratch_types=[
            pltpu.VMEM((per_w,), jnp.int32),
        ] + [pltpu.VMEM((_CHUNK, d), jnp.float32) for _ in range(_NBUF)]
          + [pltpu.SemaphoreType.DMA for _ in range(2 * _NBUF)],
    )
    def gather(table_hbm, idx_hbm, out_hbm, idx_v, *bufs_and_sems):
        bufs = bufs_and_sems[:_NBUF]
        g_sem = bufs_and_sems[_NBUF:2 * _NBUF]
        w_sem = bufs_and_sems[2 * _NBUF:]
        wid = lax.axis_index("s") * _NC + lax.axis_index("c")
        base = wid * per_w
        pltpu.sync_copy(idx_hbm.at[pl.ds(base, per_w)], idx_v)

        gcp = [None] * n_chunks
        wcp = [None] * n_chunks

        def start_gather(c):
            b = c % _NBUF
            gcp[c] = pltpu.async_copy(
                table_hbm.at[idx_v.at[pl.ds(c * _CHUNK, _CHUNK)]],
                bufs[b], g_sem[b])

        for c in range(min(_NBUF, n_chunks)):
            start_gather(c)
        waited = [False] * n_chunks
        for c in range(n_chunks):
            b = c % _NBUF
            gcp[c].wait()
            wcp[c] = pltpu.async_copy(
                bufs[b], out_hbm.at[pl.ds(base + c * _CHUNK, _CHUNK)],
                w_sem[b])
            if c + _NBUF < n_chunks:
                wcp[c].wait()
                waited[c] = True
                start_gather(c + _NBUF)
        for c in range(n_chunks):
            if not waited[c]:
                wcp[c].wait()

    return gather


def kernel(input, codebook):
    batch_shape = input.shape[:-1]
    d = input.shape[-1]
    flat = input.reshape(-1, d)
    n = flat.shape[0]
    a2 = jnp.sum(flat * flat, axis=-1, keepdims=True)
    b2 = jnp.sum(codebook * codebook, axis=-1)[None, :]
    idx_flat = _tc_argmin(flat, codebook, a2, b2)
    embed = _make_sc_gather(n, d)(codebook, idx_flat)
    return embed.reshape(*batch_shape, d), idx_flat.reshape(batch_shape)
